# baseline (speedup 1.0000x reference)
import functools

import jax
import jax.numpy as jnp
from jax import lax
from jax.experimental import pallas as pl
from jax.experimental.pallas import tpu as pltpu
from jax.experimental.pallas import tpu_sc as plsc

N = 10000
E = 320000
D = 128
NEG_SLOPE = 0.2

NP = 10240
NPT = NP // 16
EPT = 20480
NCHUNK = EPT // 128
EP = 16 * EPT


def _prep_body(x_ref, wt_ref, al_ref, ar_ref, feat_ref, el_ref, er_ref,
               mx_ref):
    i = pl.program_id(0)
    f = jnp.dot(x_ref[...], wt_ref[...], preferred_element_type=jnp.float32)
    feat_ref[...] = f
    el = jnp.dot(f, al_ref[...], preferred_element_type=jnp.float32)
    er = jnp.dot(f, ar_ref[...], preferred_element_type=jnp.float32)
    el_ref[...] = el
    er_ref[...] = er
    m = jnp.concatenate(
        [jnp.full((1, 128), jnp.max(el), jnp.float32),
         jnp.full((1, 128), jnp.max(er), jnp.float32)], axis=0)

    @pl.when(i == 0)
    def _():
        mx_ref[...] = m

    @pl.when(i > 0)
    def _():
        mx_ref[...] = jnp.maximum(mx_ref[...], m)


def _prep(x_p, wt, al, ar):
    return pl.pallas_call(
        _prep_body,
        grid=(NP // 128,),
        in_specs=[
            pl.BlockSpec((128, 128), lambda i: (i, 0)),
            pl.BlockSpec((128, 128), lambda i: (0, 0)),
            pl.BlockSpec((128, 1), lambda i: (0, 0)),
            pl.BlockSpec((128, 1), lambda i: (0, 0)),
        ],
        out_specs=[
            pl.BlockSpec((128, 128), lambda i: (i, 0)),
            pl.BlockSpec((128, 1), lambda i: (i, 0)),
            pl.BlockSpec((128, 1), lambda i: (i, 0)),
            pl.BlockSpec((2, 128), lambda i: (0, 0)),
        ],
        out_shape=[
            jax.ShapeDtypeStruct((NP, 128), jnp.float32),
            jax.ShapeDtypeStruct((NP, 1), jnp.float32),
            jax.ShapeDtypeStruct((NP, 1), jnp.float32),
            jax.ShapeDtypeStruct((2, 128), jnp.float32),
        ],
    )(x_p, wt, al, ar)


def _make_sc():
    mesh = plsc.VectorSubcoreMesh(core_axis_name="c", subcore_axis_name="s")

    @functools.partial(
        pl.kernel,
        out_type=(
            jax.ShapeDtypeStruct((2, 16, NPT), jnp.float32),
            jax.ShapeDtypeStruct((2, 16, NPT, 64), jnp.float32),
        ),
        mesh=mesh,
        compiler_params=pltpu.CompilerParams(needs_layout_passes=False,
                                             use_tc_tiling_on_sc=False),
        scratch_types=[
            pltpu.VMEM((NCHUNK, 128), jnp.int32),
            pltpu.VMEM((NCHUNK, 128), jnp.int32),
            pltpu.VMEM((NCHUNK, 128), jnp.float32),
            pltpu.VMEM((128, 64), jnp.float32),
            pltpu.VMEM((128, 64), jnp.float32),
            pltpu.VMEM((2, 128), jnp.float32),
            pltpu.VMEM((4, 128), jnp.float32),
            pltpu.VMEM_SHARED((NP,), jnp.float32),
            pltpu.VMEM_SHARED((NP, 64), jnp.float32),
            pltpu.SemaphoreType.DMA,
            pltpu.SemaphoreType.DMA,
            pltpu.SemaphoreType.DMA,
            pltpu.SemaphoreType.DMA,
            pltpu.SemaphoreType.DMA,
            pltpu.SemaphoreType.DMA,
            pltpu.SemaphoreType.DMA,
            pltpu.SemaphoreType.DMA,
            pltpu.SemaphoreType.DMA,
            pltpu.SemaphoreType.DMA,
        ],
    )
    def sc_kernel(el_hbm, er_hbm, src_hbm, dst_hbm, feat2_hbm, mx_hbm,
                  denom_out, rst_out,
                  src_v, dst_v, ee_v, rows0, rows1, mv, erow,
                  denom_sh, rst_sh,
                  semg0, semg1, sems0, sems1, semel,
                  semer0, semer1, semer2, semer3, semd):
        c = lax.axis_index("c")
        s = lax.axis_index("s")

        pltpu.sync_copy(src_hbm.at[s], src_v)
        pltpu.sync_copy(dst_hbm.at[s], dst_v)
        pltpu.sync_copy(mx_hbm, mv)

        def _zerow(i, _):
            erow[0, pl.ds(i * 16, 16)] = jnp.zeros((16,), jnp.float32)
            return 0
        lax.fori_loop(0, 8, _zerow, 0)

        def _zrow(i, _):
            r = i // 4
            cb = lax.rem(i, 4)
            rows0[r, pl.ds(cb * 16, 16)] = jnp.zeros((16,), jnp.float32)
            return 0
        lax.fori_loop(0, 128 * 4, _zrow, 0)

        for k in range(NPT // 128):
            pltpu.sync_copy(erow.at[0],
                            denom_sh.at[pl.ds(s * NPT + k * 128, 128)])
            pltpu.sync_copy(rows0, rst_sh.at[pl.ds(s * NPT + k * 128, 128)])
        plsc.subcore_barrier()

        def _mx8(row):
            def body(k, acc):
                return jnp.maximum(acc, mv[row, pl.ds(k * 16, 16)])
            return jnp.max(lax.fori_loop(1, 8, body, mv[row, pl.ds(0, 16)]))
        M = jnp.maximum(_mx8(0) + _mx8(1), 0.0)

        semer = (semer0, semer1, semer2, semer3)

        def _g_el(j):
            pltpu.async_copy(el_hbm.at[src_v.at[j]], ee_v.at[j], semel)

        def _g_er(j, b):
            pltpu.async_copy(er_hbm.at[dst_v.at[j]], erow.at[b], semer[b])

        for b in range(4):
            _g_el(b)
            _g_er(b, b)

        def _att_one(j, b):
            pltpu.make_async_copy(el_hbm.at[pl.ds(0, 128)], ee_v.at[0],
                                  semel).wait()
            pltpu.make_async_copy(er_hbm.at[pl.ds(0, 128)], erow.at[b],
                                  semer[b]).wait()
            for k in range(8):
                sl = pl.ds(k * 16, 16)
                e = ee_v[j, sl] + erow[b, sl]
                e = jnp.where(e > 0, e, NEG_SLOPE * e)
                ee_v[j, sl] = jnp.exp(e - M)
            pltpu.async_copy(ee_v.at[j], denom_sh.at[dst_v.at[j]], semd,
                             add=True)

        def att_body(t, _):
            j = 4 * t
            for b in range(4):
                _att_one(j + b, b)

                @pl.when(t < NCHUNK // 4 - 1)
                def _(b=b):
                    _g_el(j + 4 + b)
                    _g_er(j + 4 + b, b)
            return 0
        lax.fori_loop(0, NCHUNK // 4, att_body, 0)

        def src2_body(r, _):
            for k in range(8):
                cl = pl.ds(k * 16, 16)
                src_v[r, cl] = src_v[r, cl] * 2 + c
            return 0
        lax.fori_loop(0, NCHUNK, src2_body, 0)

        def _g_rows(j, buf, sem):
            pltpu.async_copy(feat2_hbm.at[src_v.at[j]], buf, sem)

        def _w_rows(buf, sem):
            pltpu.make_async_copy(feat2_hbm.at[pl.ds(0, 128)], buf,
                                  sem).wait()

        def _sc_rows(j, buf, sem):
            pltpu.async_copy(buf, rst_sh.at[dst_v.at[j]], sem, add=True)

        def _w_scat(buf, sem):
            pltpu.make_async_copy(buf, rst_sh.at[pl.ds(0, 128)], sem).wait()

        def _scale(j, buf):
            def row_body(i, _):
                r = i * 4
                for dr in range(4):
                    ar = plsc.load_gather(
                        ee_v, [jnp.full((16,), j, jnp.int32),
                               jnp.full((16,), r + dr, jnp.int32)])
                    for cb in range(4):
                        sl = pl.ds(cb * 16, 16)
                        buf[r + dr, sl] = buf[r + dr, sl] * ar
                return 0
            lax.fori_loop(0, 32, row_body, 0)

        _g_rows(0, rows0, semg0)
        _g_rows(1, rows1, semg1)

        def msg_body(t, _):
            j = 2 * t
            _w_rows(rows0, semg0)
            _scale(j, rows0)
            _sc_rows(j, rows0, sems0)
            _w_rows(rows1, semg1)
            _scale(j + 1, rows1)
            _sc_rows(j + 1, rows1, sems1)

            @pl.when(t < NCHUNK // 2 - 1)
            def _():
                _w_scat(rows0, sems0)
                _g_rows(j + 2, rows0, semg0)
                _w_scat(rows1, sems1)
                _g_rows(j + 3, rows1, semg1)
            return 0
        lax.fori_loop(0, NCHUNK // 2, msg_body, 0)
        _w_scat(rows0, sems0)
        _w_scat(rows1, sems1)

        def dr_body(j, _):
            pltpu.make_async_copy(ee_v.at[0], denom_sh.at[pl.ds(0, 128)],
                                  semd).wait()
            return 0
        lax.fori_loop(0, NCHUNK, dr_body, 0)

        plsc.subcore_barrier()

        pltpu.sync_copy(denom_sh.at[pl.ds(s * NPT, NPT)], denom_out.at[c, s])
        pltpu.sync_copy(rst_sh.at[pl.ds(s * NPT, NPT)], rst_out.at[c, s])

    return sc_kernel


_sc_kernel = _make_sc()


def _post_body(r_ref, d_ref, out_ref):
    d = d_ref[...]
    inv = jnp.where(d > 0, 1.0 / d, 0.0)
    out_ref[...] = jnp.concatenate(
        [r_ref[0] * inv, r_ref[1] * inv], axis=1)


def _post(rst_part, denom0):
    return pl.pallas_call(
        _post_body,
        grid=(NP // 128,),
        in_specs=[
            pl.BlockSpec((2, 128, 64), lambda i: (0, i, 0)),
            pl.BlockSpec((128, 1), lambda i: (i, 0)),
        ],
        out_specs=pl.BlockSpec((128, 128), lambda i: (i, 0)),
        out_shape=jax.ShapeDtypeStruct((NP, 128), jnp.float32),
    )(rst_part, denom0)


def kernel(x, edge_index, W, attn_l, attn_r):
    x_p = jnp.pad(x, ((0, NP - N), (0, 0)))
    wt = W.T
    al = attn_l.reshape(D, 1)
    ar = attn_r.reshape(D, 1)

    feat, el, er, mx = _prep(x_p, wt, al, ar)
    el1 = el.reshape(NP)
    er1 = er.reshape(NP)
    feat2 = feat.reshape(2 * NP, 64)

    src = edge_index[0]
    dst = edge_index[1]
    pad = EP - E
    src_p = jnp.concatenate([src, jnp.zeros((pad,), jnp.int32)]).reshape(
        16, NCHUNK, 128)
    dst_p = jnp.concatenate([dst, jnp.full((pad,), N, jnp.int32)]).reshape(
        16, NCHUNK, 128)

    denom_part, rst_part = _sc_kernel(el1, er1, src_p, dst_p, feat2, mx)

    rst = _post(rst_part.reshape(2, NP, 64),
                denom_part.reshape(2, NP, 1)[0])
    return rst[:N].reshape(N, 1, D)


# speedup vs baseline: 1.4062x; 1.4062x over previous
import functools

import jax
import jax.numpy as jnp
from jax import lax
from jax.experimental import pallas as pl
from jax.experimental.pallas import tpu as pltpu
from jax.experimental.pallas import tpu_sc as plsc

N = 10000
E = 320000
D = 128
NEG_SLOPE = 0.2

NP = 10240
NPT = NP // 16
EPT = 20224
NCHUNK = EPT // 128
EP = 16 * EPT


def _prep_body(x_ref, wt_ref, al_ref, ar_ref, feat_ref, el_ref, er_ref,
               mx_ref):
    i = pl.program_id(0)
    f = jnp.dot(x_ref[...], wt_ref[...], preferred_element_type=jnp.float32)
    feat_ref[...] = f
    el = jnp.dot(f, al_ref[...], preferred_element_type=jnp.float32)
    er = jnp.dot(f, ar_ref[...], preferred_element_type=jnp.float32)
    el_ref[...] = el
    er_ref[...] = er
    m = jnp.concatenate(
        [jnp.full((1, 128), jnp.max(el), jnp.float32),
         jnp.full((1, 128), jnp.max(er), jnp.float32)], axis=0)

    @pl.when(i == 0)
    def _():
        mx_ref[...] = m

    @pl.when(i > 0)
    def _():
        mx_ref[...] = jnp.maximum(mx_ref[...], m)


def _prep(x_p, wt, al, ar):
    return pl.pallas_call(
        _prep_body,
        grid=(NP // 1024,),
        in_specs=[
            pl.BlockSpec((1024, 128), lambda i: (i, 0)),
            pl.BlockSpec((128, 128), lambda i: (0, 0)),
            pl.BlockSpec((128, 1), lambda i: (0, 0)),
            pl.BlockSpec((128, 1), lambda i: (0, 0)),
        ],
        out_specs=[
            pl.BlockSpec((1024, 128), lambda i: (i, 0)),
            pl.BlockSpec((1024, 1), lambda i: (i, 0)),
            pl.BlockSpec((1024, 1), lambda i: (i, 0)),
            pl.BlockSpec((2, 128), lambda i: (0, 0)),
        ],
        out_shape=[
            jax.ShapeDtypeStruct((NP, 128), jnp.float32),
            jax.ShapeDtypeStruct((NP, 1), jnp.float32),
            jax.ShapeDtypeStruct((NP, 1), jnp.float32),
            jax.ShapeDtypeStruct((2, 128), jnp.float32),
        ],
    )(x_p, wt, al, ar)


def _make_sc():
    mesh = plsc.VectorSubcoreMesh(core_axis_name="c", subcore_axis_name="s")

    @functools.partial(
        pl.kernel,
        out_type=(
            jax.ShapeDtypeStruct((2, 16, NPT), jnp.float32),
            jax.ShapeDtypeStruct((2, 16, NPT, 64), jnp.float32),
        ),
        mesh=mesh,
        compiler_params=pltpu.CompilerParams(needs_layout_passes=False,
                                             use_tc_tiling_on_sc=False),
        scratch_types=[
            pltpu.VMEM((NCHUNK, 128), jnp.int32),
            pltpu.VMEM((NCHUNK, 128), jnp.int32),
            pltpu.VMEM((NCHUNK, 128), jnp.float32),
            pltpu.VMEM((128, 64), jnp.float32),
            pltpu.VMEM((128, 64), jnp.float32),
            pltpu.VMEM((2, 128), jnp.float32),
            pltpu.VMEM((4, 128), jnp.float32),
            pltpu.VMEM_SHARED((NP,), jnp.float32),
            pltpu.VMEM_SHARED((NP, 64), jnp.float32),
            pltpu.SemaphoreType.DMA,
            pltpu.SemaphoreType.DMA,
            pltpu.SemaphoreType.DMA,
            pltpu.SemaphoreType.DMA,
            pltpu.SemaphoreType.DMA,
            pltpu.SemaphoreType.DMA,
            pltpu.SemaphoreType.DMA,
            pltpu.SemaphoreType.DMA,
            pltpu.SemaphoreType.DMA,
            pltpu.SemaphoreType.DMA,
        ],
    )
    def sc_kernel(el_hbm, er_hbm, src_hbm, dst_hbm, feat2_hbm, mx_hbm,
                  denom_out, rst_out,
                  src_v, dst_v, ee_v, rows0, rows1, mv, erow,
                  denom_sh, rst_sh,
                  semg0, semg1, sems0, sems1, semel,
                  semer0, semer1, semer2, semer3, semd):
        c = lax.axis_index("c")
        s = lax.axis_index("s")

        pltpu.sync_copy(src_hbm.at[s], src_v)
        pltpu.sync_copy(dst_hbm.at[s], dst_v)
        pltpu.sync_copy(mx_hbm, mv)

        def _zerow(i, _):
            erow[0, pl.ds(i * 16, 16)] = jnp.zeros((16,), jnp.float32)
            return 0
        lax.fori_loop(0, 8, _zerow, 0)

        def _zrow(i, _):
            r = i // 4
            cb = lax.rem(i, 4)
            rows0[r, pl.ds(cb * 16, 16)] = jnp.zeros((16,), jnp.float32)
            return 0
        lax.fori_loop(0, 128 * 4, _zrow, 0)

        for k in range(NPT // 128):
            pltpu.sync_copy(erow.at[0],
                            denom_sh.at[pl.ds(s * NPT + k * 128, 128)])
            pltpu.sync_copy(rows0, rst_sh.at[pl.ds(s * NPT + k * 128, 128)])
        plsc.subcore_barrier()

        def _mx8(row):
            def body(k, acc):
                return jnp.maximum(acc, mv[row, pl.ds(k * 16, 16)])
            return jnp.max(lax.fori_loop(1, 8, body, mv[row, pl.ds(0, 16)]))
        M = jnp.maximum(_mx8(0) + _mx8(1), 0.0)

        def _g_el(j):
            pltpu.async_copy(el_hbm.at[src_v.at[j]], ee_v.at[j], semel)

        def _g_er(j, b, sem):
            pltpu.async_copy(er_hbm.at[dst_v.at[j]], erow.at[b], sem)

        _g_el(0)
        _g_er(0, 0, semer0)
        _g_el(1)
        _g_er(1, 1, semer1)

        def _att_one(j, b, sem):
            pltpu.make_async_copy(el_hbm.at[pl.ds(0, 128)], ee_v.at[0],
                                  semel).wait()
            pltpu.make_async_copy(er_hbm.at[pl.ds(0, 128)], erow.at[b],
                                  sem).wait()
            for k in range(8):
                sl = pl.ds(k * 16, 16)
                e = ee_v[j, sl] + erow[b, sl]
                e = jnp.where(e > 0, e, NEG_SLOPE * e)
                ee_v[j, sl] = jnp.exp(e - M)
            pltpu.async_copy(ee_v.at[j], denom_sh.at[dst_v.at[j]], semd,
                             add=True)

        def att_body(t, _):
            j = 2 * t
            _att_one(j, 0, semer0)

            @pl.when(t < NCHUNK // 2 - 1)
            def _():
                _g_el(j + 2)
                _g_er(j + 2, 0, semer0)
            _att_one(j + 1, 1, semer1)

            @pl.when(t < NCHUNK // 2 - 1)
            def _():
                _g_el(j + 3)
                _g_er(j + 3, 1, semer1)
            return 0
        lax.fori_loop(0, NCHUNK // 2, att_body, 0)

        def src2_body(i, _):
            r = i // 8
            cl = pl.ds(lax.rem(i, 8) * 16, 16)
            src_v[r, cl] = src_v[r, cl] * 2 + c
            return 0
        lax.fori_loop(0, EPT // 16, src2_body, 0)

        def _g_rows(j, buf, sem):
            pltpu.async_copy(feat2_hbm.at[src_v.at[j]], buf, sem)

        def _w_rows(buf, sem):
            pltpu.make_async_copy(feat2_hbm.at[pl.ds(0, 128)], buf,
                                  sem).wait()

        def _sc_rows(j, buf, sem):
            pltpu.async_copy(buf, rst_sh.at[dst_v.at[j]], sem, add=True)

        def _w_scat(buf, sem):
            pltpu.make_async_copy(buf, rst_sh.at[pl.ds(0, 128)], sem).wait()

        def _scale(j, buf):
            def row_body(r, _):
                ar = plsc.load_gather(
                    ee_v, [jnp.full((16,), j, jnp.int32),
                           jnp.full((16,), r, jnp.int32)])
                for cb in range(4):
                    sl = pl.ds(cb * 16, 16)
                    buf[r, sl] = buf[r, sl] * ar
                return 0
            lax.fori_loop(0, 128, row_body, 0)

        _g_rows(0, rows0, semg0)
        _g_rows(1, rows1, semg1)

        def msg_body(t, _):
            j = 2 * t
            _w_rows(rows0, semg0)
            _scale(j, rows0)
            _sc_rows(j, rows0, sems0)
            _w_rows(rows1, semg1)
            _scale(j + 1, rows1)
            _sc_rows(j + 1, rows1, sems1)

            @pl.when(t < NCHUNK // 2 - 1)
            def _():
                _w_scat(rows0, sems0)
                _g_rows(j + 2, rows0, semg0)
                _w_scat(rows1, sems1)
                _g_rows(j + 3, rows1, semg1)
            return 0
        lax.fori_loop(0, NCHUNK // 2, msg_body, 0)
        _w_scat(rows0, sems0)
        _w_scat(rows1, sems1)

        def dr_body(j, _):
            pltpu.make_async_copy(ee_v.at[0], denom_sh.at[pl.ds(0, 128)],
                                  semd).wait()
            return 0
        lax.fori_loop(0, NCHUNK, dr_body, 0)

        plsc.subcore_barrier()

        pltpu.sync_copy(denom_sh.at[pl.ds(s * NPT, NPT)], denom_out.at[c, s])
        pltpu.sync_copy(rst_sh.at[pl.ds(s * NPT, NPT)], rst_out.at[c, s])

    return sc_kernel


_sc_kernel = _make_sc()


def _post_body(r_ref, d_ref, out_ref):
    d = d_ref[...]
    inv = jnp.where(d > 0, 1.0 / d, 0.0)
    out_ref[...] = jnp.concatenate(
        [r_ref[0] * inv, r_ref[1] * inv], axis=1)


def _post(rst_part, denom0):
    return pl.pallas_call(
        _post_body,
        grid=(NP // 1024,),
        in_specs=[
            pl.BlockSpec((2, 1024, 64), lambda i: (0, i, 0)),
            pl.BlockSpec((1024, 1), lambda i: (i, 0)),
        ],
        out_specs=pl.BlockSpec((1024, 128), lambda i: (i, 0)),
        out_shape=jax.ShapeDtypeStruct((NP, 128), jnp.float32),
    )(rst_part, denom0)


def kernel(x, edge_index, W, attn_l, attn_r):
    x_p = jnp.pad(x, ((0, NP - N), (0, 0)))
    wt = W.T
    al = attn_l.reshape(D, 1)
    ar = attn_r.reshape(D, 1)

    feat, el, er, mx = _prep(x_p, wt, al, ar)
    el1 = el.reshape(NP)
    er1 = er.reshape(NP)
    feat2 = feat.reshape(2 * NP, 64)

    src = edge_index[0]
    dst = edge_index[1]
    pad = EP - E
    src_p = jnp.concatenate([src, jnp.zeros((pad,), jnp.int32)]).reshape(
        16, NCHUNK, 128)
    dst_p = jnp.concatenate([dst, jnp.full((pad,), N, jnp.int32)]).reshape(
        16, NCHUNK, 128)

    denom_part, rst_part = _sc_kernel(el1, er1, src_p, dst_p, feat2, mx)

    rst = _post(rst_part.reshape(2, NP, 64),
                denom_part.reshape(2, NP, 1)[0])
    return rst[:N].reshape(N, 1, D)


# speedup vs baseline: 1.6035x; 1.1403x over previous
import functools

import jax
import jax.numpy as jnp
from jax import lax
from jax.experimental import pallas as pl
from jax.experimental.pallas import tpu as pltpu
from jax.experimental.pallas import tpu_sc as plsc

N = 10000
E = 320000
D = 128
NEG_SLOPE = 0.2

NP = 10240
NPT = NP // 16
EPT = 20224
NCHUNK = EPT // 128
EP = 16 * EPT


def _prep_body(x_ref, wt_ref, al_ref, ar_ref, feat_ref, el_ref, er_ref,
               mx_ref):
    i = pl.program_id(0)
    f = jnp.dot(x_ref[...], wt_ref[...], preferred_element_type=jnp.float32)
    feat_ref[...] = f
    el = jnp.dot(f, al_ref[...], preferred_element_type=jnp.float32)
    er = jnp.dot(f, ar_ref[...], preferred_element_type=jnp.float32)
    el_ref[...] = el
    er_ref[...] = er
    m = jnp.concatenate(
        [jnp.full((1, 128), jnp.max(el), jnp.float32),
         jnp.full((1, 128), jnp.max(er), jnp.float32)], axis=0)

    @pl.when(i == 0)
    def _():
        mx_ref[...] = m

    @pl.when(i > 0)
    def _():
        mx_ref[...] = jnp.maximum(mx_ref[...], m)


def _prep(x_p, wt, al, ar):
    return pl.pallas_call(
        _prep_body,
        grid=(NP // 1024,),
        in_specs=[
            pl.BlockSpec((1024, 128), lambda i: (i, 0)),
            pl.BlockSpec((128, 128), lambda i: (0, 0)),
            pl.BlockSpec((128, 1), lambda i: (0, 0)),
            pl.BlockSpec((128, 1), lambda i: (0, 0)),
        ],
        out_specs=[
            pl.BlockSpec((1024, 128), lambda i: (i, 0)),
            pl.BlockSpec((1024, 1), lambda i: (i, 0)),
            pl.BlockSpec((1024, 1), lambda i: (i, 0)),
            pl.BlockSpec((2, 128), lambda i: (0, 0)),
        ],
        out_shape=[
            jax.ShapeDtypeStruct((NP, 128), jnp.float32),
            jax.ShapeDtypeStruct((NP, 1), jnp.float32),
            jax.ShapeDtypeStruct((NP, 1), jnp.float32),
            jax.ShapeDtypeStruct((2, 128), jnp.float32),
        ],
    )(x_p, wt, al, ar)


def _make_sc():
    mesh = plsc.VectorSubcoreMesh(core_axis_name="c", subcore_axis_name="s")

    @functools.partial(
        pl.kernel,
        out_type=(
            jax.ShapeDtypeStruct((2, 16, NPT), jnp.float32),
            jax.ShapeDtypeStruct((2, 16, NPT, 64), jnp.float32),
        ),
        mesh=mesh,
        compiler_params=pltpu.CompilerParams(needs_layout_passes=False,
                                             use_tc_tiling_on_sc=False),
        scratch_types=[
            pltpu.VMEM((NCHUNK, 128), jnp.int32),
            pltpu.VMEM((NCHUNK, 128), jnp.int32),
            pltpu.VMEM((NCHUNK, 128), jnp.float32),
            pltpu.VMEM((128, 64), jnp.float32),
            pltpu.VMEM((128, 64), jnp.float32),
            pltpu.VMEM((2, 128), jnp.float32),
            pltpu.VMEM((4, 128), jnp.float32),
            pltpu.VMEM_SHARED((NP,), jnp.float32),
            pltpu.VMEM_SHARED((NP, 64), jnp.float32),
            pltpu.SemaphoreType.DMA,
            pltpu.SemaphoreType.DMA,
            pltpu.SemaphoreType.DMA,
            pltpu.SemaphoreType.DMA,
            pltpu.SemaphoreType.DMA,
            pltpu.SemaphoreType.DMA,
            pltpu.SemaphoreType.DMA,
            pltpu.SemaphoreType.DMA,
            pltpu.SemaphoreType.DMA,
            pltpu.SemaphoreType.DMA,
        ],
    )
    def sc_kernel(el_hbm, er_hbm, src_hbm, dst_hbm, feat2_hbm, mx_hbm,
                  denom_out, rst_out,
                  src_v, dst_v, ee_v, rows0, rows1, mv, erow,
                  denom_sh, rst_sh,
                  semg0, semg1, sems0, sems1, semel,
                  semer0, semer1, semer2, semer3, semd):
        c = lax.axis_index("c")
        s = lax.axis_index("s")

        pltpu.sync_copy(src_hbm.at[s], src_v)
        pltpu.sync_copy(dst_hbm.at[s], dst_v)
        pltpu.sync_copy(mx_hbm, mv)

        def _zerow(i, _):
            erow[0, pl.ds(i * 16, 16)] = jnp.zeros((16,), jnp.float32)
            return 0
        lax.fori_loop(0, 8, _zerow, 0)

        def _zrow(i, _):
            r = i // 4
            cb = lax.rem(i, 4)
            rows0[r, pl.ds(cb * 16, 16)] = jnp.zeros((16,), jnp.float32)
            return 0
        lax.fori_loop(0, 128 * 4, _zrow, 0)

        for k in range(NPT // 128):
            pltpu.sync_copy(erow.at[0],
                            denom_sh.at[pl.ds(s * NPT + k * 128, 128)])
            pltpu.sync_copy(rows0, rst_sh.at[pl.ds(s * NPT + k * 128, 128)])
        plsc.subcore_barrier()

        def _mx8(row):
            def body(k, acc):
                return jnp.maximum(acc, mv[row, pl.ds(k * 16, 16)])
            return jnp.max(lax.fori_loop(1, 8, body, mv[row, pl.ds(0, 16)]))
        M = jnp.maximum(_mx8(0) + _mx8(1), 0.0)

        def _g_el(j):
            pltpu.async_copy(el_hbm.at[src_v.at[j]], ee_v.at[j], semel)

        def _g_er(j, b, sem):
            pltpu.async_copy(er_hbm.at[dst_v.at[j]], erow.at[b], sem)

        _g_el(0)
        _g_er(0, 0, semer0)
        _g_el(1)
        _g_er(1, 1, semer1)

        def _att_one(j, b, sem):
            pltpu.make_async_copy(el_hbm.at[pl.ds(0, 128)], ee_v.at[0],
                                  semel).wait()
            pltpu.make_async_copy(er_hbm.at[pl.ds(0, 128)], erow.at[b],
                                  sem).wait()
            for k in range(8):
                sl = pl.ds(k * 16, 16)
                e = ee_v[j, sl] + erow[b, sl]
                e = jnp.where(e > 0, e, NEG_SLOPE * e)
                ee_v[j, sl] = jnp.exp(e - M)
            pltpu.async_copy(ee_v.at[j], denom_sh.at[dst_v.at[j]], semd,
                             add=True)

        def att_body(t, _):
            j = 2 * t
            _att_one(j, 0, semer0)

            @pl.when(t < NCHUNK // 2 - 1)
            def _():
                _g_el(j + 2)
                _g_er(j + 2, 0, semer0)
            _att_one(j + 1, 1, semer1)

            @pl.when(t < NCHUNK // 2 - 1)
            def _():
                _g_el(j + 3)
                _g_er(j + 3, 1, semer1)
            return 0
        lax.fori_loop(0, NCHUNK // 2, att_body, 0)

        def src2_body(i, _):
            r = i // 8
            cl = pl.ds(lax.rem(i, 8) * 16, 16)
            src_v[r, cl] = src_v[r, cl] * 2 + c
            return 0
        lax.fori_loop(0, EPT // 16, src2_body, 0)

        def _g_rows(j, buf, sem):
            pltpu.async_copy(feat2_hbm.at[src_v.at[j]], buf, sem)

        def _w_rows(buf, sem):
            pltpu.make_async_copy(feat2_hbm.at[pl.ds(0, 128)], buf,
                                  sem).wait()

        def _sc_rows(j, buf, sem):
            pltpu.async_copy(buf, rst_sh.at[dst_v.at[j]], sem, add=True)

        def _w_scat(buf, sem):
            pltpu.make_async_copy(buf, rst_sh.at[pl.ds(0, 128)], sem).wait()

        def _scale(j, buf):
            @plsc.parallel_loop(0, 128, unroll=4)
            def row_body(r):
                ar = plsc.load_gather(
                    ee_v, [jnp.full((16,), j, jnp.int32),
                           jnp.full((16,), r, jnp.int32)])
                for cb in range(4):
                    sl = pl.ds(cb * 16, 16)
                    buf[r, sl] = buf[r, sl] * ar

        _g_rows(0, rows0, semg0)
        _g_rows(1, rows1, semg1)

        def msg_body(t, _):
            j = 2 * t
            _w_rows(rows0, semg0)
            _scale(j, rows0)
            _sc_rows(j, rows0, sems0)
            _w_rows(rows1, semg1)
            _scale(j + 1, rows1)
            _sc_rows(j + 1, rows1, sems1)

            @pl.when(t < NCHUNK // 2 - 1)
            def _():
                _w_scat(rows0, sems0)
                _g_rows(j + 2, rows0, semg0)
                _w_scat(rows1, sems1)
                _g_rows(j + 3, rows1, semg1)
            return 0
        lax.fori_loop(0, NCHUNK // 2, msg_body, 0)
        _w_scat(rows0, sems0)
        _w_scat(rows1, sems1)

        def dr_body(j, _):
            pltpu.make_async_copy(ee_v.at[0], denom_sh.at[pl.ds(0, 128)],
                                  semd).wait()
            return 0
        lax.fori_loop(0, NCHUNK, dr_body, 0)

        plsc.subcore_barrier()

        pltpu.sync_copy(denom_sh.at[pl.ds(s * NPT, NPT)], denom_out.at[c, s])
        pltpu.sync_copy(rst_sh.at[pl.ds(s * NPT, NPT)], rst_out.at[c, s])

    return sc_kernel


_sc_kernel = _make_sc()


def _post_body(r_ref, d_ref, out_ref):
    d = d_ref[...]
    inv = jnp.where(d > 0, 1.0 / d, 0.0)
    out_ref[...] = jnp.concatenate(
        [r_ref[0] * inv, r_ref[1] * inv], axis=1)


def _post(rst_part, denom0):
    return pl.pallas_call(
        _post_body,
        grid=(NP // 1024,),
        in_specs=[
            pl.BlockSpec((2, 1024, 64), lambda i: (0, i, 0)),
            pl.BlockSpec((1024, 1), lambda i: (i, 0)),
        ],
        out_specs=pl.BlockSpec((1024, 128), lambda i: (i, 0)),
        out_shape=jax.ShapeDtypeStruct((NP, 128), jnp.float32),
    )(rst_part, denom0)


def kernel(x, edge_index, W, attn_l, attn_r):
    x_p = jnp.pad(x, ((0, NP - N), (0, 0)))
    wt = W.T
    al = attn_l.reshape(D, 1)
    ar = attn_r.reshape(D, 1)

    feat, el, er, mx = _prep(x_p, wt, al, ar)
    el1 = el.reshape(NP)
    er1 = er.reshape(NP)
    feat2 = feat.reshape(2 * NP, 64)

    src = edge_index[0]
    dst = edge_index[1]
    pad = EP - E
    src_p = jnp.concatenate([src, jnp.zeros((pad,), jnp.int32)]).reshape(
        16, NCHUNK, 128)
    dst_p = jnp.concatenate([dst, jnp.full((pad,), N, jnp.int32)]).reshape(
        16, NCHUNK, 128)

    denom_part, rst_part = _sc_kernel(el1, er1, src_p, dst_p, feat2, mx)

    rst = _post(rst_part.reshape(2, NP, 64),
                denom_part.reshape(2, NP, 1)[0])
    return rst[:N].reshape(N, 1, D)


# speedup vs baseline: 1.6101x; 1.0041x over previous
import functools

import jax
import jax.numpy as jnp
from jax import lax
from jax.experimental import pallas as pl
from jax.experimental.pallas import tpu as pltpu
from jax.experimental.pallas import tpu_sc as plsc

N = 10000
E = 320000
D = 128
NEG_SLOPE = 0.2

NP = 10240
NPT = NP // 16
EPT = 20224
NCHUNK = EPT // 128
EP = 16 * EPT


def _prep_body(x_ref, wt_ref, al_ref, ar_ref, feat_ref, el_ref, er_ref,
               mx_ref):
    i = pl.program_id(0)
    f = jnp.dot(x_ref[...], wt_ref[...], preferred_element_type=jnp.float32)
    feat_ref[...] = f
    el = jnp.dot(f, al_ref[...], preferred_element_type=jnp.float32)
    er = jnp.dot(f, ar_ref[...], preferred_element_type=jnp.float32)
    el_ref[...] = el
    er_ref[...] = er
    m = jnp.concatenate(
        [jnp.full((1, 128), jnp.max(el), jnp.float32),
         jnp.full((1, 128), jnp.max(er), jnp.float32)], axis=0)

    @pl.when(i == 0)
    def _():
        mx_ref[...] = m

    @pl.when(i > 0)
    def _():
        mx_ref[...] = jnp.maximum(mx_ref[...], m)


def _prep(x_p, wt, al, ar):
    return pl.pallas_call(
        _prep_body,
        grid=(NP // 1024,),
        in_specs=[
            pl.BlockSpec((1024, 128), lambda i: (i, 0)),
            pl.BlockSpec((128, 128), lambda i: (0, 0)),
            pl.BlockSpec((128, 1), lambda i: (0, 0)),
            pl.BlockSpec((128, 1), lambda i: (0, 0)),
        ],
        out_specs=[
            pl.BlockSpec((1024, 128), lambda i: (i, 0)),
            pl.BlockSpec((1024, 1), lambda i: (i, 0)),
            pl.BlockSpec((1024, 1), lambda i: (i, 0)),
            pl.BlockSpec((2, 128), lambda i: (0, 0)),
        ],
        out_shape=[
            jax.ShapeDtypeStruct((NP, 128), jnp.float32),
            jax.ShapeDtypeStruct((NP, 1), jnp.float32),
            jax.ShapeDtypeStruct((NP, 1), jnp.float32),
            jax.ShapeDtypeStruct((2, 128), jnp.float32),
        ],
    )(x_p, wt, al, ar)


def _make_sc():
    mesh = plsc.VectorSubcoreMesh(core_axis_name="c", subcore_axis_name="s")

    @functools.partial(
        pl.kernel,
        out_type=(
            jax.ShapeDtypeStruct((2, 16, NPT), jnp.float32),
            jax.ShapeDtypeStruct((2, 16, NPT, 64), jnp.float32),
        ),
        mesh=mesh,
        compiler_params=pltpu.CompilerParams(needs_layout_passes=False,
                                             use_tc_tiling_on_sc=False),
        scratch_types=[
            pltpu.VMEM((NCHUNK, 128), jnp.int32),
            pltpu.VMEM((NCHUNK, 128), jnp.int32),
            pltpu.VMEM((NCHUNK, 128), jnp.float32),
            pltpu.VMEM((128, 64), jnp.float32),
            pltpu.VMEM((128, 64), jnp.float32),
            pltpu.VMEM((2, 128), jnp.float32),
            pltpu.VMEM((4, 128), jnp.float32),
            pltpu.VMEM_SHARED((NP,), jnp.float32),
            pltpu.VMEM_SHARED((NP, 64), jnp.float32),
            pltpu.SemaphoreType.DMA,
            pltpu.SemaphoreType.DMA,
            pltpu.SemaphoreType.DMA,
            pltpu.SemaphoreType.DMA,
            pltpu.SemaphoreType.DMA,
            pltpu.SemaphoreType.DMA,
            pltpu.SemaphoreType.DMA,
            pltpu.SemaphoreType.DMA,
            pltpu.SemaphoreType.DMA,
            pltpu.SemaphoreType.DMA,
        ],
    )
    def sc_kernel(el_hbm, er_hbm, src_hbm, dst_hbm, feat2_hbm, mx_hbm,
                  denom_out, rst_out,
                  src_v, dst_v, ee_v, rows0, rows1, mv, erow,
                  denom_sh, rst_sh,
                  semg0, semg1, sems0, sems1, semel,
                  semer0, semer1, semer2, semer3, semd):
        c = lax.axis_index("c")
        s = lax.axis_index("s")

        pltpu.sync_copy(src_hbm.at[s], src_v)
        pltpu.sync_copy(dst_hbm.at[s], dst_v)
        pltpu.sync_copy(mx_hbm, mv)

        def _zerow(i, _):
            erow[0, pl.ds(i * 16, 16)] = jnp.zeros((16,), jnp.float32)
            return 0
        lax.fori_loop(0, 8, _zerow, 0)

        def _zrow(i, _):
            r = i // 4
            cb = lax.rem(i, 4)
            rows0[r, pl.ds(cb * 16, 16)] = jnp.zeros((16,), jnp.float32)
            return 0
        lax.fori_loop(0, 128 * 4, _zrow, 0)

        for k in range(NPT // 128):
            pltpu.sync_copy(erow.at[0],
                            denom_sh.at[pl.ds(s * NPT + k * 128, 128)])
            pltpu.sync_copy(rows0, rst_sh.at[pl.ds(s * NPT + k * 128, 128)])
        plsc.subcore_barrier()

        def _mx8(row):
            def body(k, acc):
                return jnp.maximum(acc, mv[row, pl.ds(k * 16, 16)])
            return jnp.max(lax.fori_loop(1, 8, body, mv[row, pl.ds(0, 16)]))
        M = jnp.maximum(_mx8(0) + _mx8(1), 0.0)

        def _g_el(j):
            pltpu.async_copy(el_hbm.at[src_v.at[j]], ee_v.at[j], semel)

        def _g_er(j, b, sem):
            pltpu.async_copy(er_hbm.at[dst_v.at[j]], erow.at[b], sem)

        _g_el(0)
        _g_er(0, 0, semer0)
        _g_el(1)
        _g_er(1, 1, semer1)

        def _att_one(j, b, sem):
            pltpu.make_async_copy(el_hbm.at[pl.ds(0, 128)], ee_v.at[0],
                                  semel).wait()
            pltpu.make_async_copy(er_hbm.at[pl.ds(0, 128)], erow.at[b],
                                  sem).wait()
            for k in range(8):
                sl = pl.ds(k * 16, 16)
                e = ee_v[j, sl] + erow[b, sl]
                e = jnp.where(e > 0, e, NEG_SLOPE * e)
                ee_v[j, sl] = jnp.exp(e - M)
            pltpu.async_copy(ee_v.at[j], denom_sh.at[dst_v.at[j]], semd,
                             add=True)

        def att_body(t, _):
            j = 2 * t
            _att_one(j, 0, semer0)

            @pl.when(t < NCHUNK // 2 - 1)
            def _():
                _g_el(j + 2)
                _g_er(j + 2, 0, semer0)
            _att_one(j + 1, 1, semer1)

            @pl.when(t < NCHUNK // 2 - 1)
            def _():
                _g_el(j + 3)
                _g_er(j + 3, 1, semer1)
            return 0
        lax.fori_loop(0, NCHUNK // 2, att_body, 0)

        @plsc.parallel_loop(0, EPT // 16, unroll=4)
        def src2_body(i):
            r = i // 8
            cl = pl.ds(lax.rem(i, 8) * 16, 16)
            src_v[r, cl] = src_v[r, cl] * 2 + c

        def _g_rows(j, buf, sem):
            pltpu.async_copy(feat2_hbm.at[src_v.at[j]], buf, sem)

        def _w_rows(buf, sem):
            pltpu.make_async_copy(feat2_hbm.at[pl.ds(0, 128)], buf,
                                  sem).wait()

        def _sc_rows(j, buf, sem):
            pltpu.async_copy(buf, rst_sh.at[dst_v.at[j]], sem, add=True)

        def _w_scat(buf, sem):
            pltpu.make_async_copy(buf, rst_sh.at[pl.ds(0, 128)], sem).wait()

        def _scale(j, buf):
            @plsc.parallel_loop(0, 128, unroll=8)
            def row_body(r):
                ar = plsc.load_gather(
                    ee_v, [jnp.full((16,), j, jnp.int32),
                           jnp.full((16,), r, jnp.int32)])
                for cb in range(4):
                    sl = pl.ds(cb * 16, 16)
                    buf[r, sl] = buf[r, sl] * ar

        _g_rows(0, rows0, semg0)
        _g_rows(1, rows1, semg1)

        def msg_body(t, _):
            j = 2 * t
            _w_rows(rows0, semg0)
            _scale(j, rows0)
            _sc_rows(j, rows0, sems0)
            _w_rows(rows1, semg1)
            _scale(j + 1, rows1)
            _sc_rows(j + 1, rows1, sems1)

            @pl.when(t < NCHUNK // 2 - 1)
            def _():
                _w_scat(rows0, sems0)
                _g_rows(j + 2, rows0, semg0)
                _w_scat(rows1, sems1)
                _g_rows(j + 3, rows1, semg1)
            return 0
        lax.fori_loop(0, NCHUNK // 2, msg_body, 0)
        _w_scat(rows0, sems0)
        _w_scat(rows1, sems1)

        def dr_body(j, _):
            pltpu.make_async_copy(ee_v.at[0], denom_sh.at[pl.ds(0, 128)],
                                  semd).wait()
            return 0
        lax.fori_loop(0, NCHUNK, dr_body, 0)

        plsc.subcore_barrier()

        pltpu.sync_copy(denom_sh.at[pl.ds(s * NPT, NPT)], denom_out.at[c, s])
        pltpu.sync_copy(rst_sh.at[pl.ds(s * NPT, NPT)], rst_out.at[c, s])

    return sc_kernel


_sc_kernel = _make_sc()


def _post_body(r_ref, d_ref, out_ref):
    d = d_ref[...]
    inv = jnp.where(d > 0, 1.0 / d, 0.0)
    out_ref[...] = jnp.concatenate(
        [r_ref[0] * inv, r_ref[1] * inv], axis=1)


def _post(rst_part, denom0):
    return pl.pallas_call(
        _post_body,
        grid=(NP // 1024,),
        in_specs=[
            pl.BlockSpec((2, 1024, 64), lambda i: (0, i, 0)),
            pl.BlockSpec((1024, 1), lambda i: (i, 0)),
        ],
        out_specs=pl.BlockSpec((1024, 128), lambda i: (i, 0)),
        out_shape=jax.ShapeDtypeStruct((NP, 128), jnp.float32),
    )(rst_part, denom0)


def kernel(x, edge_index, W, attn_l, attn_r):
    x_p = jnp.pad(x, ((0, NP - N), (0, 0)))
    wt = W.T
    al = attn_l.reshape(D, 1)
    ar = attn_r.reshape(D, 1)

    feat, el, er, mx = _prep(x_p, wt, al, ar)
    el1 = el.reshape(NP)
    er1 = er.reshape(NP)
    feat2 = feat.reshape(2 * NP, 64)

    src = edge_index[0]
    dst = edge_index[1]
    pad = EP - E
    src_p = jnp.concatenate([src, jnp.zeros((pad,), jnp.int32)]).reshape(
        16, NCHUNK, 128)
    dst_p = jnp.concatenate([dst, jnp.full((pad,), N, jnp.int32)]).reshape(
        16, NCHUNK, 128)

    denom_part, rst_part = _sc_kernel(el1, er1, src_p, dst_p, feat2, mx)

    rst = _post(rst_part.reshape(2, NP, 64),
                denom_part.reshape(2, NP, 1)[0])
    return rst[:N].reshape(N, 1, D)


# speedup vs baseline: 1.6109x; 1.0005x over previous
import functools

import jax
import jax.numpy as jnp
from jax import lax
from jax.experimental import pallas as pl
from jax.experimental.pallas import tpu as pltpu
from jax.experimental.pallas import tpu_sc as plsc

N = 10000
E = 320000
D = 128
NEG_SLOPE = 0.2

NP = 10240
NPT = NP // 16
EPT = 20224
NCHUNK = EPT // 128
EP = 16 * EPT


def _prep_body(x_ref, wt_ref, al_ref, ar_ref, feat_ref, el_ref, er_ref,
               mx_ref):
    i = pl.program_id(0)
    f = jnp.dot(x_ref[...], wt_ref[...], preferred_element_type=jnp.float32)
    feat_ref[...] = f
    el = jnp.dot(f, al_ref[...], preferred_element_type=jnp.float32)
    er = jnp.dot(f, ar_ref[...], preferred_element_type=jnp.float32)
    el_ref[...] = el
    er_ref[...] = er
    m = jnp.concatenate(
        [jnp.full((1, 128), jnp.max(el), jnp.float32),
         jnp.full((1, 128), jnp.max(er), jnp.float32)], axis=0)

    @pl.when(i == 0)
    def _():
        mx_ref[...] = m

    @pl.when(i > 0)
    def _():
        mx_ref[...] = jnp.maximum(mx_ref[...], m)


def _prep(x_p, wt, al, ar):
    return pl.pallas_call(
        _prep_body,
        grid=(NP // 1024,),
        in_specs=[
            pl.BlockSpec((1024, 128), lambda i: (i, 0)),
            pl.BlockSpec((128, 128), lambda i: (0, 0)),
            pl.BlockSpec((128, 1), lambda i: (0, 0)),
            pl.BlockSpec((128, 1), lambda i: (0, 0)),
        ],
        out_specs=[
            pl.BlockSpec((1024, 128), lambda i: (i, 0)),
            pl.BlockSpec((1024, 1), lambda i: (i, 0)),
            pl.BlockSpec((1024, 1), lambda i: (i, 0)),
            pl.BlockSpec((2, 128), lambda i: (0, 0)),
        ],
        out_shape=[
            jax.ShapeDtypeStruct((NP, 128), jnp.float32),
            jax.ShapeDtypeStruct((NP, 1), jnp.float32),
            jax.ShapeDtypeStruct((NP, 1), jnp.float32),
            jax.ShapeDtypeStruct((2, 128), jnp.float32),
        ],
    )(x_p, wt, al, ar)


def _make_sc():
    mesh = plsc.VectorSubcoreMesh(core_axis_name="c", subcore_axis_name="s")

    @functools.partial(
        pl.kernel,
        out_type=(
            jax.ShapeDtypeStruct((2, 16, NPT), jnp.float32),
            jax.ShapeDtypeStruct((2, 16, NPT, 64), jnp.float32),
        ),
        mesh=mesh,
        compiler_params=pltpu.CompilerParams(needs_layout_passes=False,
                                             use_tc_tiling_on_sc=False),
        scratch_types=[
            pltpu.VMEM((NCHUNK, 128), jnp.int32),
            pltpu.VMEM((NCHUNK, 128), jnp.int32),
            pltpu.VMEM((NCHUNK, 128), jnp.float32),
            pltpu.VMEM((128, 64), jnp.float32),
            pltpu.VMEM((128, 64), jnp.float32),
            pltpu.VMEM((2, 128), jnp.float32),
            pltpu.VMEM((4, 128), jnp.float32),
            pltpu.VMEM_SHARED((NP,), jnp.float32),
            pltpu.VMEM_SHARED((NP, 64), jnp.float32),
            pltpu.SemaphoreType.DMA,
            pltpu.SemaphoreType.DMA,
            pltpu.SemaphoreType.DMA,
            pltpu.SemaphoreType.DMA,
            pltpu.SemaphoreType.DMA,
            pltpu.SemaphoreType.DMA,
            pltpu.SemaphoreType.DMA,
            pltpu.SemaphoreType.DMA,
            pltpu.SemaphoreType.DMA,
            pltpu.SemaphoreType.DMA,
        ],
    )
    def sc_kernel(el_hbm, er_hbm, src_hbm, dst_hbm, feat2_hbm, mx_hbm,
                  denom_out, rst_out,
                  src_v, dst_v, ee_v, rows0, rows1, mv, erow,
                  denom_sh, rst_sh,
                  semg0, semg1, sems0, sems1, semel,
                  semer0, semer1, semer2, semer3, semd):
        c = lax.axis_index("c")
        s = lax.axis_index("s")

        pltpu.sync_copy(src_hbm.at[s], src_v)
        pltpu.sync_copy(dst_hbm.at[s], dst_v)
        pltpu.sync_copy(mx_hbm, mv)

        def _zerow(i, _):
            erow[0, pl.ds(i * 16, 16)] = jnp.zeros((16,), jnp.float32)
            return 0
        lax.fori_loop(0, 8, _zerow, 0)

        @plsc.parallel_loop(0, 128 * 4, unroll=4)
        def _zrow(i):
            r = i // 4
            cb = lax.rem(i, 4)
            rows0[r, pl.ds(cb * 16, 16)] = jnp.zeros((16,), jnp.float32)

        for k in range(NPT // 128):
            pltpu.sync_copy(erow.at[0],
                            denom_sh.at[pl.ds(s * NPT + k * 128, 128)])
            pltpu.sync_copy(rows0, rst_sh.at[pl.ds(s * NPT + k * 128, 128)])
        plsc.subcore_barrier()

        def _mx8(row):
            def body(k, acc):
                return jnp.maximum(acc, mv[row, pl.ds(k * 16, 16)])
            return jnp.max(lax.fori_loop(1, 8, body, mv[row, pl.ds(0, 16)]))
        M = jnp.maximum(_mx8(0) + _mx8(1), 0.0)

        def _g_el(j):
            pltpu.async_copy(el_hbm.at[src_v.at[j]], ee_v.at[j], semel)

        def _g_er(j, b, sem):
            pltpu.async_copy(er_hbm.at[dst_v.at[j]], erow.at[b], sem)

        _g_el(0)
        _g_er(0, 0, semer0)
        _g_el(1)
        _g_er(1, 1, semer1)

        def _att_one(j, b, sem):
            pltpu.make_async_copy(el_hbm.at[pl.ds(0, 128)], ee_v.at[0],
                                  semel).wait()
            pltpu.make_async_copy(er_hbm.at[pl.ds(0, 128)], erow.at[b],
                                  sem).wait()
            for k in range(8):
                sl = pl.ds(k * 16, 16)
                e = ee_v[j, sl] + erow[b, sl]
                e = jnp.where(e > 0, e, NEG_SLOPE * e)
                ee_v[j, sl] = jnp.exp(e - M)
            pltpu.async_copy(ee_v.at[j], denom_sh.at[dst_v.at[j]], semd,
                             add=True)

        def att_body(t, _):
            j = 2 * t
            _att_one(j, 0, semer0)

            @pl.when(t < NCHUNK // 2 - 1)
            def _():
                _g_el(j + 2)
                _g_er(j + 2, 0, semer0)
            _att_one(j + 1, 1, semer1)

            @pl.when(t < NCHUNK // 2 - 1)
            def _():
                _g_el(j + 3)
                _g_er(j + 3, 1, semer1)
            return 0
        lax.fori_loop(0, NCHUNK // 2, att_body, 0)

        @plsc.parallel_loop(0, EPT // 16, unroll=4)
        def src2_body(i):
            r = i // 8
            cl = pl.ds(lax.rem(i, 8) * 16, 16)
            src_v[r, cl] = src_v[r, cl] * 2 + c

        def _g_rows(j, buf, sem):
            pltpu.async_copy(feat2_hbm.at[src_v.at[j]], buf, sem)

        def _w_rows(buf, sem):
            pltpu.make_async_copy(feat2_hbm.at[pl.ds(0, 128)], buf,
                                  sem).wait()

        def _sc_rows(j, buf, sem):
            pltpu.async_copy(buf, rst_sh.at[dst_v.at[j]], sem, add=True)

        def _w_scat(buf, sem):
            pltpu.make_async_copy(buf, rst_sh.at[pl.ds(0, 128)], sem).wait()

        def _scale(j, buf):
            @plsc.parallel_loop(0, 128, unroll=16)
            def row_body(r):
                ar = plsc.load_gather(
                    ee_v, [jnp.full((16,), j, jnp.int32),
                           jnp.full((16,), r, jnp.int32)])
                for cb in range(4):
                    sl = pl.ds(cb * 16, 16)
                    buf[r, sl] = buf[r, sl] * ar

        _g_rows(0, rows0, semg0)
        _g_rows(1, rows1, semg1)

        def msg_body(t, _):
            j = 2 * t
            _w_rows(rows0, semg0)
            _scale(j, rows0)
            _sc_rows(j, rows0, sems0)
            _w_rows(rows1, semg1)
            _scale(j + 1, rows1)
            _sc_rows(j + 1, rows1, sems1)

            @pl.when(t < NCHUNK // 2 - 1)
            def _():
                _w_scat(rows0, sems0)
                _g_rows(j + 2, rows0, semg0)
                _w_scat(rows1, sems1)
                _g_rows(j + 3, rows1, semg1)
            return 0
        lax.fori_loop(0, NCHUNK // 2, msg_body, 0)
        _w_scat(rows0, sems0)
        _w_scat(rows1, sems1)

        def dr_body(j, _):
            pltpu.make_async_copy(ee_v.at[0], denom_sh.at[pl.ds(0, 128)],
                                  semd).wait()
            return 0
        lax.fori_loop(0, NCHUNK, dr_body, 0)

        plsc.subcore_barrier()

        pltpu.sync_copy(denom_sh.at[pl.ds(s * NPT, NPT)], denom_out.at[c, s])
        pltpu.sync_copy(rst_sh.at[pl.ds(s * NPT, NPT)], rst_out.at[c, s])

    return sc_kernel


_sc_kernel = _make_sc()


def _post_body(r_ref, d_ref, out_ref):
    d = d_ref[...]
    inv = jnp.where(d > 0, 1.0 / d, 0.0)
    out_ref[...] = jnp.concatenate(
        [r_ref[0] * inv, r_ref[1] * inv], axis=1)


def _post(rst_part, denom0):
    return pl.pallas_call(
        _post_body,
        grid=(NP // 1024,),
        in_specs=[
            pl.BlockSpec((2, 1024, 64), lambda i: (0, i, 0)),
            pl.BlockSpec((1024, 1), lambda i: (i, 0)),
        ],
        out_specs=pl.BlockSpec((1024, 128), lambda i: (i, 0)),
        out_shape=jax.ShapeDtypeStruct((NP, 128), jnp.float32),
    )(rst_part, denom0)


def kernel(x, edge_index, W, attn_l, attn_r):
    x_p = jnp.pad(x, ((0, NP - N), (0, 0)))
    wt = W.T
    al = attn_l.reshape(D, 1)
    ar = attn_r.reshape(D, 1)

    feat, el, er, mx = _prep(x_p, wt, al, ar)
    el1 = el.reshape(NP)
    er1 = er.reshape(NP)
    feat2 = feat.reshape(2 * NP, 64)

    src = edge_index[0]
    dst = edge_index[1]
    pad = EP - E
    src_p = jnp.concatenate([src, jnp.zeros((pad,), jnp.int32)]).reshape(
        16, NCHUNK, 128)
    dst_p = jnp.concatenate([dst, jnp.full((pad,), N, jnp.int32)]).reshape(
        16, NCHUNK, 128)

    denom_part, rst_part = _sc_kernel(el1, er1, src_p, dst_p, feat2, mx)

    rst = _post(rst_part.reshape(2, NP, 64),
                denom_part.reshape(2, NP, 1)[0])
    return rst[:N].reshape(N, 1, D)
